# baseline (speedup 1.0000x reference)
import functools

import jax
import jax.numpy as jnp
from jax.experimental import pallas as pl
from jax.experimental.pallas import tpu as pltpu

UNITS = 64
MAX_N = 3
SHF = 16
RBF = MAX_N * SHF
CUTOFF = 5.0

ROWS = 4096


def _sig(x):
    return jax.nn.sigmoid(x)


def _silu(x):
    return x * _sig(x)


def _dot(a, b):
    return jax.lax.dot_general(a, b, (((1,), (0,)), ((), ())),
                               preferred_element_type=jnp.float32)


def _basis_body(v1_ref, v2_ref, tb_ref):
    v1 = v1_ref[...]
    v2 = v2_ref[...]
    d2 = jnp.sum(v2 * v2, axis=1, keepdims=True)
    tlen = jnp.sqrt(d2)
    n1 = v1 / (jnp.sqrt(jnp.sum(v1 * v1, axis=1, keepdims=True)) + 1e-8)
    n2 = v2 / (tlen + 1e-8)
    x = jnp.clip(jnp.sum(n1 * n2, axis=1, keepdims=True), -1.0, 1.0)
    r = tlen
    coef = jnp.sqrt(2.0 / CUTOFF)
    s = jnp.pi / CUTOFF
    inv = coef / (r + 1e-8)
    rad = [jnp.sin((s * n) * r) * inv for n in (1.0, 2.0, 3.0)]
    ang = [jnp.ones_like(x), x]
    for _ in range(SHF - 2):
        ang.append(2.0 * x * ang[-1] - ang[-2])
    a16 = jnp.concatenate(ang, axis=1)
    tb_ref[...] = jnp.concatenate([rn * a16 for rn in rad], axis=1)


def _basis(v1, v2):
    t = v1.shape[0]
    grid = pl.cdiv(t, ROWS)
    return pl.pallas_call(
        _basis_body,
        grid=(grid,),
        in_specs=[
            pl.BlockSpec((ROWS, 3), lambda i: (i, 0)),
            pl.BlockSpec((ROWS, 3), lambda i: (i, 0)),
        ],
        out_specs=pl.BlockSpec((ROWS, RBF), lambda i: (i, 0)),
        out_shape=jax.ShapeDtypeStruct((t, RBF), jnp.float32),
    )(v1, v2)


def _bond_body(agg_ref, bond_ref, aS_ref, aR_ref,
               wl_ref, bl_ref, wg_ref, bg_ref,
               wb1_ref, wb2_ref, wb3_ref, bb_ref,
               wa_ref, ba_ref, wag_ref, bag_ref,
               bond_out, m_out):
    agg = agg_ref[...]
    bond = bond_ref[...]
    g1 = _silu(_dot(agg, wl_ref[...]) + bl_ref[...])
    g2 = _sig(_dot(agg, wg_ref[...]) + bg_ref[...])
    bond1 = bond + g1 * g2
    pre = (_dot(aS_ref[...], wb1_ref[...]) + _dot(aR_ref[...], wb2_ref[...])
           + _dot(bond1, wb3_ref[...]) + bb_ref[...])
    bond2 = bond1 + _silu(pre)
    m = _silu(_dot(bond2, wa_ref[...]) + ba_ref[...]) * _sig(
        _dot(bond2, wag_ref[...]) + bag_ref[...])
    bond_out[...] = bond2
    m_out[...] = m


def _bond_block(agg, bond, aS, aR, blk):
    e = bond.shape[0]
    grid = pl.cdiv(e, ROWS)
    wb = blk['Wb']
    row = pl.BlockSpec((ROWS, None), lambda i: (i, 0))
    full = lambda r, c: pl.BlockSpec((r, c), lambda i: (0, 0))
    return pl.pallas_call(
        _bond_body,
        grid=(grid,),
        in_specs=[
            pl.BlockSpec((ROWS, RBF), lambda i: (i, 0)),
            pl.BlockSpec((ROWS, UNITS), lambda i: (i, 0)),
            pl.BlockSpec((ROWS, UNITS), lambda i: (i, 0)),
            pl.BlockSpec((ROWS, UNITS), lambda i: (i, 0)),
            full(RBF, UNITS), full(1, UNITS),
            full(RBF, UNITS), full(1, UNITS),
            full(UNITS, UNITS), full(UNITS, UNITS), full(UNITS, UNITS),
            full(1, UNITS),
            full(UNITS, UNITS), full(1, UNITS),
            full(UNITS, UNITS), full(1, UNITS),
        ],
        out_specs=[
            pl.BlockSpec((ROWS, UNITS), lambda i: (i, 0)),
            pl.BlockSpec((ROWS, UNITS), lambda i: (i, 0)),
        ],
        out_shape=[
            jax.ShapeDtypeStruct((e, UNITS), jnp.float32),
            jax.ShapeDtypeStruct((e, UNITS), jnp.float32),
        ],
    )(agg, bond, aS, aR,
      blk['Wl'], blk['bl'][None, :], blk['Wg'], blk['bg'][None, :],
      wb[:UNITS], wb[UNITS:2 * UNITS], wb[2 * UNITS:], blk['bb'][None, :],
      blk['Wa'], blk['ba'][None, :], blk['Wag'], blk['bag'][None, :])


def kernel(atom_positions, atom_types, bond_atom_indices,
           triple_bond_indices, batch_ids, params):
    pos = atom_positions
    n = pos.shape[0]
    e = bond_atom_indices.shape[0]
    sender = bond_atom_indices[:, 0]
    receiver = bond_atom_indices[:, 1]
    vec = pos[receiver] - pos[sender]
    blen = jnp.sqrt(jnp.sum(vec * vec, axis=1, keepdims=True))
    b1 = triple_bond_indices[:, 0]
    b2 = triple_bond_indices[:, 1]
    v1 = vec[b1]
    v2 = vec[b2]
    tb = _basis(v1, v2)
    apex = receiver[b2]
    atom = params['emb'][atom_types]
    bond = _silu(blen @ params['Wbp'] + params['bbp'])
    for blk in params['blocks']:
        upd = _sig(atom @ blk['Wu'] + blk['bu'])
        msg = tb * upd[apex]
        agg = jax.ops.segment_sum(msg, b1, num_segments=e)
        bond, m = _bond_block(agg, bond, atom[sender], atom[receiver], blk)
        atom = atom + jax.ops.segment_sum(m, receiver, num_segments=n)
    per_atom = atom @ params['Wr'] + params['br']
    energy = jax.ops.segment_sum(per_atom, batch_ids,
                                 num_segments=128)
    return energy
